# baseline (speedup 1.0000x reference)
import functools

import jax
import jax.numpy as jnp
from jax import lax
from jax.experimental import pallas as pl
from jax.experimental.pallas import tpu as pltpu
from jax.experimental.pallas import tpu_sc as plsc

_VOCAB = 1000000
_DIM = 64
_BATCH = 16384
_NNEG = 81920

_NC, _NS = 2, 16
_NW = _NC * _NS
_PACK = _VOCAB // 2
_VTAIL = (_VOCAB // 128) * 128
_PTAIL = _VTAIL // 2
_WCOL = 512
_NWIN = _VTAIL // _WCOL
_GRP = 16
_CHUNK = 128
_P_PER = _BATCH // _NW
_N_PER = _NNEG // _NW
_P_CH = _P_PER // _CHUNK
_N_CH = _N_PER // _CHUNK


def _tp_body(in_t, out_t, in_tail, out_tail, in_lin, out_lin,
             tw0, tw1, pack0, pack1, sem0, sem1, semo0, semo1):
  wid = lax.axis_index("s") * _NC + lax.axis_index("c")
  lanes = lax.iota(jnp.int32, _GRP)
  ge8 = lax.shift_right_logical(lanes, 3)
  d8 = lax.bitwise_and(lanes, 7)
  dcols = [tg * 8 + d8 + ge8 * _DIM for tg in range(8)]

  n_my = jnp.where(wid < _NWIN % _NW, _NWIN // _NW + 1, _NWIN // _NW)
  tws = (tw0, tw1)
  sems = (sem0, sem1)
  packs = (pack0, pack1)
  semos = (semo0, semo1)
  _PROW = _WCOL // 2

  def table(src, dst, tail_src):
    def start(p, slot):
      t, tg = p // 8, p % 8
      w = wid + t * _NW
      pltpu.async_copy(src.at[pl.ds(tg * 8, 8), pl.ds(w * _WCOL, _WCOL)],
                       tws[slot], sems[slot])

    def wait_in(slot):
      pltpu.make_async_copy(src.at[pl.ds(0, 8), pl.ds(0, _WCOL)],
                            tws[slot], sems[slot]).wait()

    start(0, 0)

    def win_one(t, slot_w):
      pk = packs[slot_w]

      @pl.when(t >= 2)
      def _():
        pltpu.make_async_copy(pk, dst.at[pl.ds(0, _PROW)],
                              semos[slot_w]).wait()

      for tg in range(8):
        p = t * 8 + tg
        slot = tg % 2
        wait_in(slot)

        @pl.when(p + 1 < n_my * 8)
        def _(p=p, slot=slot):
          start(p + 1, 1 - slot)

        buf = tws[slot]
        dc = dcols[tg]

        def jp_block(b, _, buf=buf, dc=dc):
          vs = []
          for i in range(8):
            c = 2 * (b * 8 + i) + ge8
            vs.append(plsc.load_gather(buf, [d8, c]))
          for i in range(8):
            plsc.store_scatter(pk, [jnp.full((_GRP,), b * 8 + i, jnp.int32),
                                    dc], vs[i])
          return 0

        lax.fori_loop(0, _PROW // 8, jp_block, 0)

      w = wid + t * _NW
      pltpu.async_copy(pk, dst.at[pl.ds(w * _PROW, _PROW)], semos[slot_w])

    def win_pair(t2, _):
      win_one(2 * t2, 0)

      @pl.when(2 * t2 + 1 < n_my)
      def _():
        win_one(2 * t2 + 1, 1)

      return 0

    lax.fori_loop(0, (n_my + 1) // 2, win_pair, 0)

    pltpu.make_async_copy(packs[0], dst.at[pl.ds(0, _PROW)], semos[0]).wait()
    pltpu.make_async_copy(packs[1], dst.at[pl.ds(0, _PROW)], semos[1]).wait()

    @pl.when(wid == 0)
    def _():
      pltpu.sync_copy(tail_src, dst.at[pl.ds(_PTAIL, (_VOCAB - _VTAIL) // 2)])

  table(in_t, in_lin, in_tail)
  table(out_t, out_lin, out_tail)


_tp_call = functools.partial(
    pl.kernel,
    out_type=(
        jax.ShapeDtypeStruct((_PACK, 128), jnp.float32),
        jax.ShapeDtypeStruct((_PACK, 128), jnp.float32),
    ),
    mesh=plsc.VectorSubcoreMesh(core_axis_name="c", subcore_axis_name="s",
                                num_cores=_NC, num_subcores=_NS),
    compiler_params=pltpu.CompilerParams(needs_layout_passes=False,
                                         use_tc_tiling_on_sc=True),
    scratch_types=(
        pltpu.VMEM((8, _WCOL), jnp.float32),
        pltpu.VMEM((8, _WCOL), jnp.float32),
        pltpu.VMEM((_WCOL // 2, 128), jnp.float32),
        pltpu.VMEM((_WCOL // 2, 128), jnp.float32),
        pltpu.SemaphoreType.DMA,
        pltpu.SemaphoreType.DMA,
        pltpu.SemaphoreType.DMA,
        pltpu.SemaphoreType.DMA,
    ),
)(_tp_body)


def _sc_body(in_lin, out_lin, cw, ctx, negw, negc,
             ce_out, ps_out, ns_out,
             idx_pa, idx_pb, idx_na, idx_nb,
             slab_a0, slab_b0, slab_a1, slab_b1,
             a_buf0, b_buf0, a_buf1, b_buf1,
             ce_pack, ps_buf, ns_buf,
             sem_a0, sem_b0, sem_a1, sem_b1):
  wid = lax.axis_index("s") * _NC + lax.axis_index("c")
  lanes = lax.iota(jnp.int32, _GRP)
  slots = ((slab_a0, slab_b0, a_buf0, b_buf0, sem_a0, sem_b0),
           (slab_a1, slab_b1, a_buf1, b_buf1, sem_a1, sem_b1))

  pltpu.sync_copy(cw.at[pl.ds(wid * _P_PER, _P_PER)], idx_pa)
  pltpu.sync_copy(ctx.at[pl.ds(wid * _P_PER, _P_PER)], idx_pb)
  pltpu.sync_copy(negc.at[pl.ds(wid * _N_PER, _N_PER)], idx_na)
  pltpu.sync_copy(negw.at[pl.ds(wid * _N_PER, _N_PER)], idx_nb)

  def fire(c, ia, ib, slot):
    sa, sb, ab, bb, sma, smb = slots[slot]
    for k in range(_CHUNK // _GRP):
      v_a = ia[pl.ds(c * _CHUNK + k * _GRP, _GRP)]
      v_b = ib[pl.ds(c * _CHUNK + k * _GRP, _GRP)]
      sa[pl.ds(k * _GRP, _GRP)] = lax.shift_right_logical(v_a, 1)
      sb[pl.ds(k * _GRP, _GRP)] = lax.shift_right_logical(v_b, 1)
    pltpu.async_copy(in_lin.at[sa], ab, sma)
    pltpu.async_copy(out_lin.at[sb], bb, smb)

  def wait(slot):
    sa, sb, ab, bb, sma, smb = slots[slot]
    pltpu.make_async_copy(in_lin.at[sa], ab, sma).wait()
    pltpu.make_async_copy(out_lin.at[sb], bb, smb).wait()

  def compute(c, ia, ib, score_buf, with_ce, slot):
    _, _, a_buf, b_buf, _, _ = slots[slot]

    def group(g, _):
      j = g * _GRP + lanes
      va_i = ia[pl.ds(c * _CHUNK + g * _GRP, _GRP)]
      vb_i = ib[pl.ds(c * _CHUNK + g * _GRP, _GRP)]
      base_a = lax.bitwise_and(va_i, 1) * _DIM
      base_b = lax.bitwise_and(vb_i, 1) * _DIM
      if with_ce:
        jp = lax.shift_right_logical(j, 1)
        ce_col0 = lax.bitwise_and(j, 1) * _DIM

      def dstep(d, acc):
        va = plsc.load_gather(a_buf, [j, base_a + d])
        vb = plsc.load_gather(b_buf, [j, base_b + d])
        if with_ce:
          plsc.store_scatter(ce_pack, [jp, ce_col0 + d], va)
        return acc + va * vb

      acc = lax.fori_loop(0, _DIM, dstep, jnp.zeros((_GRP,), jnp.float32),
                          unroll=8)
      score_buf[pl.ds(c * _CHUNK + g * _GRP, _GRP)] = acc
      return 0

    lax.fori_loop(0, _CHUNK // _GRP, group, 0)

  def phase(n_ch, ia, ib, score_buf, with_ce, post):
    fire(0, ia, ib, 0)

    def pair(c2, _):
      c = 2 * c2

      fire(c + 1, ia, ib, 1)
      wait(0)
      compute(c, ia, ib, score_buf, with_ce, 0)
      post(c)

      @pl.when(c + 2 < n_ch)
      def _():
        fire(c + 2, ia, ib, 0)

      wait(1)
      compute(c + 1, ia, ib, score_buf, with_ce, 1)
      post(c + 1)
      return 0

    lax.fori_loop(0, n_ch // 2, pair, 0)

  def ce_post(c):
    pltpu.sync_copy(
        ce_pack,
        ce_out.at[pl.ds(wid * (_P_PER // 2) + c * (_CHUNK // 2), _CHUNK // 2)])

  phase(_P_CH, idx_pa, idx_pb, ps_buf, True, ce_post)
  pltpu.sync_copy(ps_buf, ps_out.at[pl.ds(wid * _P_PER, _P_PER)])
  phase(_N_CH, idx_na, idx_nb, ns_buf, False, lambda c: None)
  pltpu.sync_copy(ns_buf, ns_out.at[pl.ds(wid * _N_PER, _N_PER)])


_sc_call = functools.partial(
    pl.kernel,
    out_type=(
        jax.ShapeDtypeStruct((_BATCH // 2, 128), jnp.float32),
        jax.ShapeDtypeStruct((_BATCH,), jnp.float32),
        jax.ShapeDtypeStruct((_NNEG,), jnp.float32),
    ),
    mesh=plsc.VectorSubcoreMesh(core_axis_name="c", subcore_axis_name="s",
                                num_cores=_NC, num_subcores=_NS),
    compiler_params=pltpu.CompilerParams(needs_layout_passes=False,
                                         use_tc_tiling_on_sc=True),
    scratch_types=(
        pltpu.VMEM((_P_PER,), jnp.int32),
        pltpu.VMEM((_P_PER,), jnp.int32),
        pltpu.VMEM((_N_PER,), jnp.int32),
        pltpu.VMEM((_N_PER,), jnp.int32),
        pltpu.VMEM((_CHUNK,), jnp.int32),
        pltpu.VMEM((_CHUNK,), jnp.int32),
        pltpu.VMEM((_CHUNK,), jnp.int32),
        pltpu.VMEM((_CHUNK,), jnp.int32),
        pltpu.VMEM((_CHUNK, 128), jnp.float32),
        pltpu.VMEM((_CHUNK, 128), jnp.float32),
        pltpu.VMEM((_CHUNK, 128), jnp.float32),
        pltpu.VMEM((_CHUNK, 128), jnp.float32),
        pltpu.VMEM((_CHUNK // 2, 128), jnp.float32),
        pltpu.VMEM((_P_PER,), jnp.float32),
        pltpu.VMEM((_N_PER,), jnp.float32),
        pltpu.SemaphoreType.DMA,
        pltpu.SemaphoreType.DMA,
        pltpu.SemaphoreType.DMA,
        pltpu.SemaphoreType.DMA,
    ),
)(_sc_body)


def _log_sigmoid(x):
  return jnp.minimum(x, 0.0) - jnp.log1p(jnp.exp(-jnp.abs(x)))


def _loss_body(ps_ref, ns_ref, out_ref):
  pos_mean = jnp.mean(_log_sigmoid(ps_ref[...]))
  neg_mean = jnp.mean(_log_sigmoid(-ns_ref[...]))
  out_ref[0] = pos_mean
  out_ref[1] = neg_mean
  out_ref[2] = -(pos_mean + neg_mean)


_loss_call = pl.pallas_call(
    _loss_body,
    out_shape=jax.ShapeDtypeStruct((3,), jnp.float32),
    out_specs=pl.BlockSpec(memory_space=pltpu.SMEM),
)


def kernel(center_words, context_words, negative_words, negative_centers,
           in_emb, out_emb):
  cw = center_words.astype(jnp.int32)
  ctx = context_words.astype(jnp.int32)
  negw = negative_words.astype(jnp.int32)
  negc = negative_centers.astype(jnp.int32)
  in_lin = in_emb.reshape(_PACK, 128)
  out_lin = out_emb.reshape(_PACK, 128)
  ce2, ps, ns = _sc_call(in_lin, out_lin, cw, ctx, negw, negc)
  ce = ce2.reshape(_BATCH, _DIM)
  losses = _loss_call(ps.reshape(_BATCH // 128, 128),
                      ns.reshape(_NNEG // 128, 128))
  return (losses[0], ps, ce, losses[1], ns, losses[2])


# speedup vs baseline: 1.3593x; 1.3593x over previous
import functools

import jax
import jax.numpy as jnp
from jax import lax
from jax.experimental import pallas as pl
from jax.experimental.pallas import tpu as pltpu
from jax.experimental.pallas import tpu_sc as plsc

_VOCAB = 1000000
_DIM = 64
_BATCH = 16384
_NNEG = 81920

_NC, _NS = 2, 16
_NW = _NC * _NS
_PACK = _VOCAB // 2
_VTAIL = (_VOCAB // 128) * 128
_PTAIL = _VTAIL // 2
_WCOL = 512
_NWIN = _VTAIL // _WCOL
_GRP = 16
_CHUNK = 128
_P_PER = _BATCH // _NW
_N_PER = _NNEG // _NW
_P_CH = _P_PER // _CHUNK
_N_CH = _N_PER // _CHUNK


def _tp_body(in_t, out_t, in_tail, out_tail, in_lin, out_lin,
             tw0, tw1, tw2, tw3, pack0, pack1,
             sem0, sem1, sem2, sem3, semo0, semo1):
  wid = lax.axis_index("s") * _NC + lax.axis_index("c")
  lanes = lax.iota(jnp.int32, _GRP)
  ge8 = lax.shift_right_logical(lanes, 3)
  d8 = lax.bitwise_and(lanes, 7)
  dcols = [tg * 8 + d8 + ge8 * _DIM for tg in range(8)]

  n_my = jnp.where(wid < _NWIN % _NW, _NWIN // _NW + 1, _NWIN // _NW)
  tws = (tw0, tw1, tw2, tw3)
  sems = (sem0, sem1, sem2, sem3)
  packs = (pack0, pack1)
  semos = (semo0, semo1)
  _PROW = _WCOL // 2

  def table(src, dst, tail_src):
    def start(p, slot):
      t, tg = p // 8, p % 8
      w = wid + t * _NW
      pltpu.async_copy(src.at[pl.ds(tg * 8, 8), pl.ds(w * _WCOL, _WCOL)],
                       tws[slot], sems[slot])

    def wait_in(slot):
      pltpu.make_async_copy(src.at[pl.ds(0, 8), pl.ds(0, _WCOL)],
                            tws[slot], sems[slot]).wait()

    start(0, 0)
    start(1, 1)
    start(2, 2)

    def win_one(t, slot_w):
      pk = packs[slot_w]

      @pl.when(t >= 2)
      def _():
        pltpu.make_async_copy(pk, dst.at[pl.ds(0, _PROW)],
                              semos[slot_w]).wait()

      for tg in range(8):
        p = t * 8 + tg
        slot = tg % 4
        wait_in(slot)

        @pl.when(p + 3 < n_my * 8)
        def _(p=p, slot=slot):
          start(p + 3, (slot + 3) % 4)

        buf = tws[slot]
        dc = dcols[tg]

        def jp_block(b, _, buf=buf, dc=dc):
          vs = []
          for i in range(8):
            c = 2 * (b * 8 + i) + ge8
            vs.append(plsc.load_gather(buf, [d8, c]))
          for i in range(8):
            plsc.store_scatter(pk, [jnp.full((_GRP,), b * 8 + i, jnp.int32),
                                    dc], vs[i])
          return 0

        lax.fori_loop(0, _PROW // 8, jp_block, 0)

      w = wid + t * _NW
      pltpu.async_copy(pk, dst.at[pl.ds(w * _PROW, _PROW)], semos[slot_w])

    def win_pair(t2, _):
      win_one(2 * t2, 0)

      @pl.when(2 * t2 + 1 < n_my)
      def _():
        win_one(2 * t2 + 1, 1)

      return 0

    lax.fori_loop(0, (n_my + 1) // 2, win_pair, 0)

    pltpu.make_async_copy(packs[0], dst.at[pl.ds(0, _PROW)], semos[0]).wait()
    pltpu.make_async_copy(packs[1], dst.at[pl.ds(0, _PROW)], semos[1]).wait()

    @pl.when(wid == 0)
    def _():
      pltpu.sync_copy(tail_src, dst.at[pl.ds(_PTAIL, (_VOCAB - _VTAIL) // 2)])

  table(in_t, in_lin, in_tail)
  table(out_t, out_lin, out_tail)


_tp_call = functools.partial(
    pl.kernel,
    out_type=(
        jax.ShapeDtypeStruct((_PACK, 128), jnp.float32),
        jax.ShapeDtypeStruct((_PACK, 128), jnp.float32),
    ),
    mesh=plsc.VectorSubcoreMesh(core_axis_name="c", subcore_axis_name="s",
                                num_cores=_NC, num_subcores=_NS),
    compiler_params=pltpu.CompilerParams(needs_layout_passes=False,
                                         use_tc_tiling_on_sc=True),
    scratch_types=(
        pltpu.VMEM((8, _WCOL), jnp.float32),
        pltpu.VMEM((8, _WCOL), jnp.float32),
        pltpu.VMEM((8, _WCOL), jnp.float32),
        pltpu.VMEM((8, _WCOL), jnp.float32),
        pltpu.VMEM((_WCOL // 2, 128), jnp.float32),
        pltpu.VMEM((_WCOL // 2, 128), jnp.float32),
        pltpu.SemaphoreType.DMA,
        pltpu.SemaphoreType.DMA,
        pltpu.SemaphoreType.DMA,
        pltpu.SemaphoreType.DMA,
        pltpu.SemaphoreType.DMA,
        pltpu.SemaphoreType.DMA,
    ),
)(_tp_body)


def _sc_body(in_lin, out_lin, cw, ctx, negw, negc,
             ce_out, ps_out, ns_out,
             idx_pa, idx_pb, idx_na, idx_nb,
             slab_a0, slab_b0, slab_a1, slab_b1,
             a_buf0, b_buf0, a_buf1, b_buf1,
             ce_pack, ps_buf, ns_buf,
             sem_a0, sem_b0, sem_a1, sem_b1):
  wid = lax.axis_index("s") * _NC + lax.axis_index("c")
  lanes = lax.iota(jnp.int32, _GRP)
  slots = ((slab_a0, slab_b0, a_buf0, b_buf0, sem_a0, sem_b0),
           (slab_a1, slab_b1, a_buf1, b_buf1, sem_a1, sem_b1))

  pltpu.sync_copy(cw.at[pl.ds(wid * _P_PER, _P_PER)], idx_pa)
  pltpu.sync_copy(ctx.at[pl.ds(wid * _P_PER, _P_PER)], idx_pb)
  pltpu.sync_copy(negc.at[pl.ds(wid * _N_PER, _N_PER)], idx_na)
  pltpu.sync_copy(negw.at[pl.ds(wid * _N_PER, _N_PER)], idx_nb)

  def fire(c, ia, ib, slot):
    sa, sb, ab, bb, sma, smb = slots[slot]
    for k in range(_CHUNK // _GRP):
      v_a = ia[pl.ds(c * _CHUNK + k * _GRP, _GRP)]
      v_b = ib[pl.ds(c * _CHUNK + k * _GRP, _GRP)]
      sa[pl.ds(k * _GRP, _GRP)] = lax.shift_right_logical(v_a, 1)
      sb[pl.ds(k * _GRP, _GRP)] = lax.shift_right_logical(v_b, 1)
    pltpu.async_copy(in_lin.at[sa], ab, sma)
    pltpu.async_copy(out_lin.at[sb], bb, smb)

  def wait(slot):
    sa, sb, ab, bb, sma, smb = slots[slot]
    pltpu.make_async_copy(in_lin.at[sa], ab, sma).wait()
    pltpu.make_async_copy(out_lin.at[sb], bb, smb).wait()

  def compute(c, ia, ib, score_buf, with_ce, slot):
    _, _, a_buf, b_buf, _, _ = slots[slot]

    def group(g, _):
      j = g * _GRP + lanes
      va_i = ia[pl.ds(c * _CHUNK + g * _GRP, _GRP)]
      vb_i = ib[pl.ds(c * _CHUNK + g * _GRP, _GRP)]
      base_a = lax.bitwise_and(va_i, 1) * _DIM
      base_b = lax.bitwise_and(vb_i, 1) * _DIM
      if with_ce:
        jp = lax.shift_right_logical(j, 1)
        ce_col0 = lax.bitwise_and(j, 1) * _DIM

      def dblock(db, acc):
        d0 = db * 8
        vas = [plsc.load_gather(a_buf, [j, base_a + (d0 + i)])
               for i in range(8)]
        vbs = [plsc.load_gather(b_buf, [j, base_b + (d0 + i)])
               for i in range(8)]
        if with_ce:
          for i in range(8):
            plsc.store_scatter(ce_pack, [jp, ce_col0 + (d0 + i)], vas[i])
        for i in range(8):
          acc = acc + vas[i] * vbs[i]
        return acc

      acc = lax.fori_loop(0, _DIM // 8, dblock,
                          jnp.zeros((_GRP,), jnp.float32))
      score_buf[pl.ds(c * _CHUNK + g * _GRP, _GRP)] = acc
      return 0

    lax.fori_loop(0, _CHUNK // _GRP, group, 0)

  def phase(n_ch, ia, ib, score_buf, with_ce, post):
    fire(0, ia, ib, 0)

    def pair(c2, _):
      c = 2 * c2

      fire(c + 1, ia, ib, 1)
      wait(0)
      compute(c, ia, ib, score_buf, with_ce, 0)
      post(c)

      @pl.when(c + 2 < n_ch)
      def _():
        fire(c + 2, ia, ib, 0)

      wait(1)
      compute(c + 1, ia, ib, score_buf, with_ce, 1)
      post(c + 1)
      return 0

    lax.fori_loop(0, n_ch // 2, pair, 0)

  def ce_post(c):
    pltpu.sync_copy(
        ce_pack,
        ce_out.at[pl.ds(wid * (_P_PER // 2) + c * (_CHUNK // 2), _CHUNK // 2)])

  phase(_P_CH, idx_pa, idx_pb, ps_buf, True, ce_post)
  pltpu.sync_copy(ps_buf, ps_out.at[pl.ds(wid * _P_PER, _P_PER)])
  phase(_N_CH, idx_na, idx_nb, ns_buf, False, lambda c: None)
  pltpu.sync_copy(ns_buf, ns_out.at[pl.ds(wid * _N_PER, _N_PER)])


_sc_call = functools.partial(
    pl.kernel,
    out_type=(
        jax.ShapeDtypeStruct((_BATCH // 2, 128), jnp.float32),
        jax.ShapeDtypeStruct((_BATCH,), jnp.float32),
        jax.ShapeDtypeStruct((_NNEG,), jnp.float32),
    ),
    mesh=plsc.VectorSubcoreMesh(core_axis_name="c", subcore_axis_name="s",
                                num_cores=_NC, num_subcores=_NS),
    compiler_params=pltpu.CompilerParams(needs_layout_passes=False,
                                         use_tc_tiling_on_sc=True),
    scratch_types=(
        pltpu.VMEM((_P_PER,), jnp.int32),
        pltpu.VMEM((_P_PER,), jnp.int32),
        pltpu.VMEM((_N_PER,), jnp.int32),
        pltpu.VMEM((_N_PER,), jnp.int32),
        pltpu.VMEM((_CHUNK,), jnp.int32),
        pltpu.VMEM((_CHUNK,), jnp.int32),
        pltpu.VMEM((_CHUNK,), jnp.int32),
        pltpu.VMEM((_CHUNK,), jnp.int32),
        pltpu.VMEM((_CHUNK, 128), jnp.float32),
        pltpu.VMEM((_CHUNK, 128), jnp.float32),
        pltpu.VMEM((_CHUNK, 128), jnp.float32),
        pltpu.VMEM((_CHUNK, 128), jnp.float32),
        pltpu.VMEM((_CHUNK // 2, 128), jnp.float32),
        pltpu.VMEM((_P_PER,), jnp.float32),
        pltpu.VMEM((_N_PER,), jnp.float32),
        pltpu.SemaphoreType.DMA,
        pltpu.SemaphoreType.DMA,
        pltpu.SemaphoreType.DMA,
        pltpu.SemaphoreType.DMA,
    ),
)(_sc_body)


def _log_sigmoid(x):
  return jnp.minimum(x, 0.0) - jnp.log1p(jnp.exp(-jnp.abs(x)))


def _loss_body(ps_ref, ns_ref, out_ref):
  pos_mean = jnp.mean(_log_sigmoid(ps_ref[...]))
  neg_mean = jnp.mean(_log_sigmoid(-ns_ref[...]))
  out_ref[0] = pos_mean
  out_ref[1] = neg_mean
  out_ref[2] = -(pos_mean + neg_mean)


_loss_call = pl.pallas_call(
    _loss_body,
    out_shape=jax.ShapeDtypeStruct((3,), jnp.float32),
    out_specs=pl.BlockSpec(memory_space=pltpu.SMEM),
)


def kernel(center_words, context_words, negative_words, negative_centers,
           in_emb, out_emb):
  cw = center_words.astype(jnp.int32)
  ctx = context_words.astype(jnp.int32)
  negw = negative_words.astype(jnp.int32)
  negc = negative_centers.astype(jnp.int32)
  in_tail = in_emb[_VTAIL:].reshape((_VOCAB - _VTAIL) // 2, 128)
  out_tail = out_emb[_VTAIL:].reshape((_VOCAB - _VTAIL) // 2, 128)
  in_lin, out_lin = _tp_call(in_emb.T, out_emb.T, in_tail, out_tail)
  ce2, ps, ns = _sc_call(in_lin, out_lin, cw, ctx, negw, negc)
  ce = ce2.reshape(_BATCH, _DIM)
  losses = _loss_call(ps.reshape(_BATCH // 128, 128),
                      ns.reshape(_NNEG // 128, 128))
  return (losses[0], ps, ce, losses[1], ns, losses[2])
